# baseline (speedup 1.0000x reference)
import jax
import jax.numpy as jnp
from jax import lax
from jax.experimental import pallas as pl
from jax.experimental.pallas import tpu as pltpu
from jax.experimental.pallas import tpu_sc as plsc

NC, NS, L = 2, 16, 16
NW = NC * NS
B = 16384
EMB = 16
BPW = B // NW
CH = 128
NFCH = BPW // CH
NECH = 3 * BPW // CH


def _sc_body(fidx_hbm, faces_hbm, emb_hbm, out_hbm,
             fidx_v, eidx_v, vidx_v, rows_v, sem, sem2):
    wid = lax.axis_index("s") * NC + lax.axis_index("c")
    base = wid * BPW

    pltpu.sync_copy(fidx_hbm.at[pl.ds(wid * NFCH, NFCH)], fidx_v)

    for c in range(BPW // L):
        f16 = fidx_v[c // 8, pl.ds((c % 8) * L, L)]
        e = f16 * 3
        for v in range(3):
            q = v * BPW + c * L
            eidx_v[q // CH, pl.ds(q % CH, L)] = e + v

    face_copies = [
        pltpu.make_async_copy(
            faces_hbm.at[eidx_v.at[j]],
            vidx_v.at[j],
            sem,
        )
        for j in range(NECH)
    ]
    for c in face_copies:
        c.start()
    for c in face_copies:
        c.wait()

    emb_copies = [
        pltpu.make_async_copy(
            emb_hbm.at[vidx_v.at[j]],
            rows_v.at[j // NFCH, pl.ds((j % NFCH) * CH, CH)],
            sem2,
        )
        for j in range(NECH)
    ]
    for c in emb_copies:
        c.start()
    for c in emb_copies:
        c.wait()

    for v in range(3):
        pltpu.sync_copy(rows_v.at[v], out_hbm.at[v, pl.ds(base, BPW)])


@jax.jit
def _sc_gather(fidx2d, faces, embeddings):
    mesh = plsc.VectorSubcoreMesh(
        core_axis_name="c", subcore_axis_name="s",
        num_cores=NC, num_subcores=NS)
    return pl.kernel(
        _sc_body,
        out_type=jax.ShapeDtypeStruct((3, B, EMB), jnp.float32),
        mesh=mesh,
        scratch_types=[
            pltpu.VMEM((NFCH, CH), jnp.int32),
            pltpu.VMEM((NECH, CH), jnp.int32),
            pltpu.VMEM((NECH, CH), jnp.int32),
            pltpu.VMEM((3, BPW, EMB), jnp.float32),
            pltpu.SemaphoreType.DMA,
            pltpu.SemaphoreType.DMA,
        ],
        compiler_params=pltpu.CompilerParams(
            needs_layout_passes=False, use_tc_tiling_on_sc=False),
    )(fidx2d, faces, embeddings)


def _mlp_body(rows_ref, bary_ref, w1_ref, b1_ref, w2_ref, b2_ref,
              w3_ref, b3_ref, o_ref):
    r = rows_ref[...]
    w = bary_ref[...]
    x = (w[:, 0:1] * r[0] + w[:, 1:2] * r[1] + w[:, 2:3] * r[2])
    h = jnp.maximum(
        jnp.dot(x, w1_ref[...], preferred_element_type=jnp.float32)
        + b1_ref[...], 0.0)
    h = jnp.maximum(
        jnp.dot(h, w2_ref[...], preferred_element_type=jnp.float32)
        + b2_ref[...], 0.0)
    z = (jnp.dot(h, w3_ref[...], preferred_element_type=jnp.float32)
         + b3_ref[...])
    o_ref[...] = jax.nn.sigmoid(z)


@jax.jit
def _tc_blend_mlp(rows, bary, W1, b1, W2, b2, W3, b3):
    nblk = 4
    blk = B // nblk
    return pl.pallas_call(
        _mlp_body,
        out_shape=jax.ShapeDtypeStruct((B, 3), jnp.float32),
        grid=(nblk,),
        in_specs=[
            pl.BlockSpec((3, blk, EMB), lambda i: (0, i, 0)),
            pl.BlockSpec((blk, 3), lambda i: (i, 0)),
            pl.BlockSpec((EMB, 32), lambda i: (0, 0)),
            pl.BlockSpec((1, 32), lambda i: (0, 0)),
            pl.BlockSpec((32, 32), lambda i: (0, 0)),
            pl.BlockSpec((1, 32), lambda i: (0, 0)),
            pl.BlockSpec((32, 3), lambda i: (0, 0)),
            pl.BlockSpec((1, 3), lambda i: (0, 0)),
        ],
        out_specs=pl.BlockSpec((blk, 3), lambda i: (i, 0)),
    )(rows, bary, W1, b1.reshape(1, 32), W2, b2.reshape(1, 32),
      W3, b3.reshape(1, 3))


def kernel(barycentrics, face_idx, faces, embeddings, W1, b1, W2, b2, W3, b3):
    fidx2d = face_idx.astype(jnp.int32).reshape(NW * NFCH, CH)
    faces_flat = faces.reshape(3 * faces.shape[0])
    rows = _sc_gather(fidx2d, faces_flat, embeddings)
    return _tc_blend_mlp(rows, barycentrics, W1, b1, W2, b2, W3, b3)


# speedup vs baseline: 3.8742x; 3.8742x over previous
import jax
import jax.numpy as jnp
from jax import lax
from jax.experimental import pallas as pl
from jax.experimental.pallas import tpu as pltpu
from jax.experimental.pallas import tpu_sc as plsc

NC, NS, L = 2, 16, 16
NW = NC * NS
B = 16384
EMB = 16
NF = 2000000
NV = 1000000
BPW = B // NW
CH = 128
NFCH = BPW // CH
NECH = 3 * BPW // CH


def _sc_body(fidx_hbm, faces_hbm, emb_hbm, out_hbm,
             fidx_v, feidx_v, vidx_v, eidx_v, rows_v, sem, sem2):
    wid = lax.axis_index("s") * NC + lax.axis_index("c")
    base = wid * BPW

    pltpu.sync_copy(fidx_hbm.at[pl.ds(wid * NFCH, NFCH)], fidx_v)

    for c in range(BPW // L):
        f16 = fidx_v[c // 8, pl.ds((c % 8) * L, L)]
        for v in range(3):
            q = v * BPW + c * L
            feidx_v[q // CH, pl.ds(q % CH, L)] = f16 + (v * NF)

    face_copies = [
        pltpu.make_async_copy(
            faces_hbm.at[feidx_v.at[j]],
            vidx_v.at[j],
            sem,
        )
        for j in range(NECH)
    ]
    for c in face_copies:
        c.start()
    for c in face_copies:
        c.wait()

    def build(j, carry):
        for k in range(CH // L):
            vv = vidx_v[j, pl.ds(k * L, L)]
            for e in range(EMB):
                eidx_v[e, j, pl.ds(k * L, L)] = vv + (e * NV)
        return carry

    lax.fori_loop(0, NECH, build, 0)

    emb_copies = [
        pltpu.make_async_copy(
            emb_hbm.at[eidx_v.at[e, j]],
            rows_v.at[e, pl.ds(j * CH, CH)],
            sem2,
        )
        for e in range(EMB)
        for j in range(NECH)
    ]
    for c in emb_copies:
        c.start()
    for c in emb_copies:
        c.wait()

    for v in range(3):
        for e in range(EMB):
            pltpu.sync_copy(rows_v.at[e, pl.ds(v * BPW, BPW)],
                            out_hbm.at[v, e, pl.ds(base, BPW)])


@jax.jit
def _sc_gather(fidx2d, faces_flat, emb_flat):
    mesh = plsc.VectorSubcoreMesh(
        core_axis_name="c", subcore_axis_name="s",
        num_cores=NC, num_subcores=NS)
    return pl.kernel(
        _sc_body,
        out_type=jax.ShapeDtypeStruct((3, EMB, B), jnp.float32),
        mesh=mesh,
        scratch_types=[
            pltpu.VMEM((NFCH, CH), jnp.int32),
            pltpu.VMEM((NECH, CH), jnp.int32),
            pltpu.VMEM((NECH, CH), jnp.int32),
            pltpu.VMEM((EMB, NECH, CH), jnp.int32),
            pltpu.VMEM((EMB, 3 * BPW), jnp.float32),
            pltpu.SemaphoreType.DMA,
            pltpu.SemaphoreType.DMA,
        ],
        compiler_params=pltpu.CompilerParams(
            needs_layout_passes=False, use_tc_tiling_on_sc=False),
    )(fidx2d, faces_flat, emb_flat)


def _mlp_body(rows_ref, bary_ref, w1_ref, b1_ref, w2_ref, b2_ref,
              w3_ref, b3_ref, o_ref):
    r = rows_ref[...]
    w = bary_ref[...]
    xt = (w[0:1, :] * r[0] + w[1:2, :] * r[1] + w[2:3, :] * r[2])
    dn = (((0,), (0,)), ((), ()))
    h = jnp.maximum(
        lax.dot_general(w1_ref[...], xt, dn,
                        preferred_element_type=jnp.float32)
        + b1_ref[...], 0.0)
    h = jnp.maximum(
        lax.dot_general(w2_ref[...], h, dn,
                        preferred_element_type=jnp.float32)
        + b2_ref[...], 0.0)
    z = (lax.dot_general(w3_ref[...], h, dn,
                         preferred_element_type=jnp.float32)
         + b3_ref[...])
    o_ref[...] = jax.nn.sigmoid(z)


@jax.jit
def _tc_blend_mlp(rows, baryT, W1, b1, W2, b2, W3, b3):
    nblk = 4
    blk = B // nblk
    return pl.pallas_call(
        _mlp_body,
        out_shape=jax.ShapeDtypeStruct((3, B), jnp.float32),
        grid=(nblk,),
        in_specs=[
            pl.BlockSpec((3, EMB, blk), lambda i: (0, 0, i)),
            pl.BlockSpec((3, blk), lambda i: (0, i)),
            pl.BlockSpec((EMB, 32), lambda i: (0, 0)),
            pl.BlockSpec((32, 1), lambda i: (0, 0)),
            pl.BlockSpec((32, 32), lambda i: (0, 0)),
            pl.BlockSpec((32, 1), lambda i: (0, 0)),
            pl.BlockSpec((32, 3), lambda i: (0, 0)),
            pl.BlockSpec((3, 1), lambda i: (0, 0)),
        ],
        out_specs=pl.BlockSpec((3, blk), lambda i: (0, i)),
    )(rows, baryT, W1, b1.reshape(32, 1), W2, b2.reshape(32, 1),
      W3, b3.reshape(3, 1))


def kernel(barycentrics, face_idx, faces, embeddings, W1, b1, W2, b2, W3, b3):
    fidx2d = face_idx.astype(jnp.int32).reshape(NW * NFCH, CH)
    faces_flat = faces.T.reshape(3 * NF)
    emb_flat = embeddings.T.reshape(EMB * NV)
    rows = _sc_gather(fidx2d, faces_flat, emb_flat)
    out_t = _tc_blend_mlp(rows, barycentrics.T, W1, b1, W2, b2, W3, b3)
    return out_t.T


# speedup vs baseline: 14.4571x; 3.7316x over previous
import jax
import jax.numpy as jnp
from jax import lax
from jax.experimental import pallas as pl
from jax.experimental.pallas import tpu as pltpu
from jax.experimental.pallas import tpu_sc as plsc

NC, NS, L = 2, 16, 16
NW = NC * NS
B = 16384
EMB = 16
BPW = B // NW
CH = 128
NFCH = BPW // CH


def _sc_body(fidx_hbm, f0_hbm, f1_hbm, f2_hbm, *rest):
    emb_hbm = rest[:EMB]
    out_hbm = rest[EMB]
    fidx_v, vidx_v, rows_v, sem, sem2 = rest[EMB + 1:]
    faces_hbm = (f0_hbm, f1_hbm, f2_hbm)

    wid = lax.axis_index("s") * NC + lax.axis_index("c")
    base = wid * BPW

    pltpu.sync_copy(fidx_hbm.at[pl.ds(wid * NFCH, NFCH)], fidx_v)

    face_copies = [
        pltpu.make_async_copy(
            faces_hbm[v].at[fidx_v.at[j]],
            vidx_v.at[v, j],
            sem,
        )
        for v in range(3)
        for j in range(NFCH)
    ]
    for c in face_copies:
        c.start()
    for c in face_copies:
        c.wait()

    emb_copies = [
        pltpu.make_async_copy(
            emb_hbm[e].at[vidx_v.at[v, j]],
            rows_v.at[e, pl.ds(v * BPW + j * CH, CH)],
            sem2,
        )
        for e in range(EMB)
        for v in range(3)
        for j in range(NFCH)
    ]
    for c in emb_copies:
        c.start()
    for c in emb_copies:
        c.wait()

    for v in range(3):
        for e in range(EMB):
            pltpu.sync_copy(rows_v.at[e, pl.ds(v * BPW, BPW)],
                            out_hbm.at[v, e, pl.ds(base, BPW)])


@jax.jit
def _sc_gather(fidx2d, f0, f1, f2, *emb_cols):
    mesh = plsc.VectorSubcoreMesh(
        core_axis_name="c", subcore_axis_name="s",
        num_cores=NC, num_subcores=NS)
    return pl.kernel(
        _sc_body,
        out_type=jax.ShapeDtypeStruct((3, EMB, B), jnp.float32),
        mesh=mesh,
        scratch_types=[
            pltpu.VMEM((NFCH, CH), jnp.int32),
            pltpu.VMEM((3, NFCH, CH), jnp.int32),
            pltpu.VMEM((EMB, 3 * BPW), jnp.float32),
            pltpu.SemaphoreType.DMA,
            pltpu.SemaphoreType.DMA,
        ],
        compiler_params=pltpu.CompilerParams(
            needs_layout_passes=False, use_tc_tiling_on_sc=False),
    )(fidx2d, f0, f1, f2, *emb_cols)


def _mlp_body(rows_ref, bary_ref, w1_ref, b1_ref, w2_ref, b2_ref,
              w3_ref, b3_ref, o_ref):
    r = rows_ref[...]
    w = bary_ref[...]
    xt = (w[0:1, :] * r[0] + w[1:2, :] * r[1] + w[2:3, :] * r[2])
    dn = (((0,), (0,)), ((), ()))
    h = jnp.maximum(
        lax.dot_general(w1_ref[...], xt, dn,
                        preferred_element_type=jnp.float32)
        + b1_ref[...], 0.0)
    h = jnp.maximum(
        lax.dot_general(w2_ref[...], h, dn,
                        preferred_element_type=jnp.float32)
        + b2_ref[...], 0.0)
    z = (lax.dot_general(w3_ref[...], h, dn,
                         preferred_element_type=jnp.float32)
         + b3_ref[...])
    o_ref[...] = jax.nn.sigmoid(z)


@jax.jit
def _tc_blend_mlp(rows, baryT, W1, b1, W2, b2, W3, b3):
    nblk = 4
    blk = B // nblk
    return pl.pallas_call(
        _mlp_body,
        out_shape=jax.ShapeDtypeStruct((3, B), jnp.float32),
        grid=(nblk,),
        in_specs=[
            pl.BlockSpec((3, EMB, blk), lambda i: (0, 0, i)),
            pl.BlockSpec((3, blk), lambda i: (0, i)),
            pl.BlockSpec((EMB, 32), lambda i: (0, 0)),
            pl.BlockSpec((32, 1), lambda i: (0, 0)),
            pl.BlockSpec((32, 32), lambda i: (0, 0)),
            pl.BlockSpec((32, 1), lambda i: (0, 0)),
            pl.BlockSpec((32, 3), lambda i: (0, 0)),
            pl.BlockSpec((3, 1), lambda i: (0, 0)),
        ],
        out_specs=pl.BlockSpec((3, blk), lambda i: (0, i)),
    )(rows, baryT, W1, b1.reshape(32, 1), W2, b2.reshape(32, 1),
      W3, b3.reshape(3, 1))


def kernel(barycentrics, face_idx, faces, embeddings, W1, b1, W2, b2, W3, b3):
    fidx2d = face_idx.astype(jnp.int32).reshape(NW * NFCH, CH)
    f_cols = [faces[:, v] for v in range(3)]
    e_cols = [embeddings[:, e] for e in range(EMB)]
    rows = _sc_gather(fidx2d, *f_cols, *e_cols)
    out_t = _tc_blend_mlp(rows, barycentrics.T, W1, b1, W2, b2, W3, b3)
    return out_t.T


# speedup vs baseline: 33.8217x; 2.3395x over previous
import jax
import jax.numpy as jnp
from jax import lax
from jax.experimental import pallas as pl
from jax.experimental.pallas import tpu as pltpu
from jax.experimental.pallas import tpu_sc as plsc

NC, NS, L = 2, 16, 16
NW = NC * NS
B = 16384
EMB = 16
BPW = B // NW
CH = 128
NFCH = BPW // CH


def _sc_body(fidx_hbm, f0_hbm, f1_hbm, f2_hbm, *rest):
    emb_hbm = rest[:EMB]
    out_hbm = rest[EMB]
    fidx_v, vidx_v, rows_v, sem, sem2 = rest[EMB + 1:]
    faces_hbm = (f0_hbm, f1_hbm, f2_hbm)

    wid = lax.axis_index("s") * NC + lax.axis_index("c")
    base = wid * BPW

    pltpu.sync_copy(fidx_hbm.at[pl.ds(wid * NFCH, NFCH)], fidx_v)

    face_copies = [
        pltpu.make_async_copy(
            faces_hbm[v].at[fidx_v.at[j]],
            vidx_v.at[v, j],
            sem,
        )
        for v in range(3)
        for j in range(NFCH)
    ]
    for c in face_copies:
        c.start()
    for c in face_copies:
        c.wait()

    emb_copies = [
        pltpu.make_async_copy(
            emb_hbm[e].at[vidx_v.at[v, j]],
            rows_v.at[e, pl.ds(v * BPW + j * CH, CH)],
            sem2,
        )
        for e in range(EMB)
        for v in range(3)
        for j in range(NFCH)
    ]
    for c in emb_copies:
        c.start()
    for c in emb_copies:
        c.wait()

    for v in range(3):
        for e in range(EMB):
            pltpu.sync_copy(rows_v.at[e, pl.ds(v * BPW, BPW)],
                            out_hbm.at[v, e, pl.ds(base, BPW)])


@jax.jit
def _sc_gather(fidx2d, f0, f1, f2, *emb_cols):
    mesh = plsc.VectorSubcoreMesh(
        core_axis_name="c", subcore_axis_name="s",
        num_cores=NC, num_subcores=NS)
    return pl.kernel(
        _sc_body,
        out_type=jax.ShapeDtypeStruct((3, EMB, B), jnp.float32),
        mesh=mesh,
        scratch_types=[
            pltpu.VMEM((NFCH, CH), jnp.int32),
            pltpu.VMEM((3, NFCH, CH), jnp.int32),
            pltpu.VMEM((EMB, 3 * BPW), jnp.float32),
            pltpu.SemaphoreType.DMA,
            pltpu.SemaphoreType.DMA,
        ],
        compiler_params=pltpu.CompilerParams(
            needs_layout_passes=False, use_tc_tiling_on_sc=False),
    )(fidx2d, f0, f1, f2, *emb_cols)


def _extract_body(src_hbm, *rest):
    outs = rest[:EMB]
    vbuf, sem = rest[EMB:]
    for r in range(2):
        pltpu.sync_copy(src_hbm.at[pl.ds(r * 8, 8)], vbuf)
        outc = [pltpu.make_async_copy(vbuf.at[k], outs[r * 8 + k], sem)
                for k in range(8)]
        for c in outc:
            c.start()
        for c in outc:
            c.wait()


@jax.jit
def _extract_emb(embT):
    NV = embT.shape[1]
    return pl.pallas_call(
        _extract_body,
        out_shape=[jax.ShapeDtypeStruct((NV,), jnp.float32)] * EMB,
        in_specs=[pl.BlockSpec(memory_space=pl.ANY)],
        out_specs=[pl.BlockSpec(memory_space=pl.ANY)] * EMB,
        scratch_shapes=[
            pltpu.VMEM((8, NV), jnp.float32),
            pltpu.SemaphoreType.DMA,
        ],
    )(embT)


def _mlp_body(rows_ref, bary_ref, w1_ref, b1_ref, w2_ref, b2_ref,
              w3_ref, b3_ref, o_ref):
    r = rows_ref[...]
    w = bary_ref[...]
    xt = (w[0:1, :] * r[0] + w[1:2, :] * r[1] + w[2:3, :] * r[2])
    dn = (((0,), (0,)), ((), ()))
    h = jnp.maximum(
        lax.dot_general(w1_ref[...], xt, dn,
                        preferred_element_type=jnp.float32)
        + b1_ref[...], 0.0)
    h = jnp.maximum(
        lax.dot_general(w2_ref[...], h, dn,
                        preferred_element_type=jnp.float32)
        + b2_ref[...], 0.0)
    z = (lax.dot_general(w3_ref[...], h, dn,
                         preferred_element_type=jnp.float32)
         + b3_ref[...])
    o_ref[...] = jax.nn.sigmoid(z)


@jax.jit
def _tc_blend_mlp(rows, baryT, W1, b1, W2, b2, W3, b3):
    nblk = 4
    blk = B // nblk
    return pl.pallas_call(
        _mlp_body,
        out_shape=jax.ShapeDtypeStruct((3, B), jnp.float32),
        grid=(nblk,),
        in_specs=[
            pl.BlockSpec((3, EMB, blk), lambda i: (0, 0, i)),
            pl.BlockSpec((3, blk), lambda i: (0, i)),
            pl.BlockSpec((EMB, 32), lambda i: (0, 0)),
            pl.BlockSpec((32, 1), lambda i: (0, 0)),
            pl.BlockSpec((32, 32), lambda i: (0, 0)),
            pl.BlockSpec((32, 1), lambda i: (0, 0)),
            pl.BlockSpec((32, 3), lambda i: (0, 0)),
            pl.BlockSpec((3, 1), lambda i: (0, 0)),
        ],
        out_specs=pl.BlockSpec((3, blk), lambda i: (0, i)),
    )(rows, baryT, W1, b1.reshape(32, 1), W2, b2.reshape(32, 1),
      W3, b3.reshape(3, 1))


def kernel(barycentrics, face_idx, faces, embeddings, W1, b1, W2, b2, W3, b3):
    fidx2d = face_idx.astype(jnp.int32).reshape(NW * NFCH, CH)
    f_cols = [faces[:, v] for v in range(3)]
    e_cols = _extract_emb(embeddings.T)
    rows = _sc_gather(fidx2d, *f_cols, *e_cols)
    out_t = _tc_blend_mlp(rows, barycentrics.T, W1, b1, W2, b2, W3, b3)
    return out_t.T


# speedup vs baseline: 57.0116x; 1.6857x over previous
import functools

import jax
import jax.numpy as jnp
from jax import lax
from jax.experimental import pallas as pl
from jax.experimental.pallas import tpu as pltpu
from jax.experimental.pallas import tpu_sc as plsc

NC, NS, L = 2, 16, 16
NW = NC * NS
B = 16384
EMB = 16
BPW = B // NW
CH = 128
NFCH = BPW // CH


def _sc_body(fidx_hbm, f0_hbm, f1_hbm, f2_hbm, *rest):
    emb_hbm = rest[:EMB]
    out_hbm = rest[EMB]
    fidx_v, vidx_v, rows_v, sem, sem2 = rest[EMB + 1:]
    faces_hbm = (f0_hbm, f1_hbm, f2_hbm)

    wid = lax.axis_index("s") * NC + lax.axis_index("c")
    base = wid * BPW

    pltpu.sync_copy(fidx_hbm.at[pl.ds(wid * NFCH, NFCH)], fidx_v)

    face_copies = [
        pltpu.make_async_copy(
            faces_hbm[v].at[fidx_v.at[j]],
            vidx_v.at[v, j],
            sem,
        )
        for v in range(3)
        for j in range(NFCH)
    ]
    for c in face_copies:
        c.start()
    for c in face_copies:
        c.wait()

    emb_copies = [
        pltpu.make_async_copy(
            emb_hbm[e].at[vidx_v.at[v, j]],
            rows_v.at[e, pl.ds(v * BPW + j * CH, CH)],
            sem2,
        )
        for e in range(EMB)
        for v in range(3)
        for j in range(NFCH)
    ]
    for c in emb_copies:
        c.start()
    for c in emb_copies:
        c.wait()

    for v in range(3):
        for e in range(EMB):
            pltpu.sync_copy(rows_v.at[e, pl.ds(v * BPW, BPW)],
                            out_hbm.at[v, e, pl.ds(base, BPW)])


@jax.jit
def _sc_gather(fidx2d, f0, f1, f2, *emb_cols):
    mesh = plsc.VectorSubcoreMesh(
        core_axis_name="c", subcore_axis_name="s",
        num_cores=NC, num_subcores=NS)
    return pl.kernel(
        _sc_body,
        out_type=jax.ShapeDtypeStruct((3, EMB, B), jnp.float32),
        mesh=mesh,
        scratch_types=[
            pltpu.VMEM((NFCH, CH), jnp.int32),
            pltpu.VMEM((3, NFCH, CH), jnp.int32),
            pltpu.VMEM((EMB, 3 * BPW), jnp.float32),
            pltpu.SemaphoreType.DMA,
            pltpu.SemaphoreType.DMA,
        ],
        compiler_params=pltpu.CompilerParams(
            needs_layout_passes=False, use_tc_tiling_on_sc=False),
    )(fidx2d, f0, f1, f2, *emb_cols)


def _chunk_plan(total):
    n = 4
    step = (total // n) // 128 * 128
    offs = [i * step for i in range(n)]
    sizes = [step] * (n - 1) + [total - (n - 1) * step]
    return list(zip(offs, sizes))


def _extract_body(rb, nrb, src_hbm, *rest):
    outs = rest[:rb * nrb]
    buf0, buf1, in0, in1, osem = rest[rb * nrb:]
    bufs, isems = (buf0, buf1), (in0, in1)
    units = [(r, off, n)
             for r in range(nrb)
             for off, n in _chunk_plan(src_hbm.shape[1])]

    def in_copy(i):
        r, off, n = units[i]
        return pltpu.make_async_copy(
            src_hbm.at[pl.ds(r * rb, rb), pl.ds(off, n)],
            bufs[i % 2].at[pl.ds(0, rb), pl.ds(0, n)],
            isems[i % 2])

    def out_copies(i):
        r, off, n = units[i]
        return [pltpu.make_async_copy(
                    bufs[i % 2].at[k, pl.ds(0, n)],
                    outs[r * rb + k].at[pl.ds(off, n)],
                    osem)
                for k in range(rb)]

    nu = len(units)
    in_copy(0).start()
    if nu > 1:
        in_copy(1).start()
    for i in range(nu):
        in_copy(i).wait()
        oc = out_copies(i)
        for c in oc:
            c.start()
        for c in oc:
            c.wait()
        if i + 2 < nu:
            in_copy(i + 2).start()


def _extract_cols(srcT, rb):
    nf, n = srcT.shape
    nrb = nf // rb
    maxn = _chunk_plan(n)[-1][1]
    return pl.pallas_call(
        functools.partial(_extract_body, rb, nrb),
        out_shape=[jax.ShapeDtypeStruct((n,), srcT.dtype)] * nf,
        in_specs=[pl.BlockSpec(memory_space=pl.ANY)],
        out_specs=[pl.BlockSpec(memory_space=pl.ANY)] * nf,
        scratch_shapes=[
            pltpu.VMEM((8, maxn), srcT.dtype),
            pltpu.VMEM((8, maxn), srcT.dtype),
            pltpu.SemaphoreType.DMA,
            pltpu.SemaphoreType.DMA,
            pltpu.SemaphoreType.DMA,
        ],
    )(srcT)


@jax.jit
def _extract_emb(embT):
    return _extract_cols(embT, 8)


@jax.jit
def _extract_faces(facesT):
    return _extract_cols(facesT, 3)


def _mlp_body(rows_ref, bary_ref, w1_ref, b1_ref, w2_ref, b2_ref,
              w3_ref, b3_ref, o_ref):
    r = rows_ref[...]
    w = bary_ref[...]
    xt = (w[0:1, :] * r[0] + w[1:2, :] * r[1] + w[2:3, :] * r[2])
    dn = (((0,), (0,)), ((), ()))
    h = jnp.maximum(
        lax.dot_general(w1_ref[...], xt, dn,
                        preferred_element_type=jnp.float32)
        + b1_ref[...], 0.0)
    h = jnp.maximum(
        lax.dot_general(w2_ref[...], h, dn,
                        preferred_element_type=jnp.float32)
        + b2_ref[...], 0.0)
    z = (lax.dot_general(w3_ref[...], h, dn,
                         preferred_element_type=jnp.float32)
         + b3_ref[...])
    o_ref[...] = jax.nn.sigmoid(z)


@jax.jit
def _tc_blend_mlp(rows, baryT, W1, b1, W2, b2, W3, b3):
    nblk = 4
    blk = B // nblk
    return pl.pallas_call(
        _mlp_body,
        out_shape=jax.ShapeDtypeStruct((3, B), jnp.float32),
        grid=(nblk,),
        in_specs=[
            pl.BlockSpec((3, EMB, blk), lambda i: (0, 0, i)),
            pl.BlockSpec((3, blk), lambda i: (0, i)),
            pl.BlockSpec((EMB, 32), lambda i: (0, 0)),
            pl.BlockSpec((32, 1), lambda i: (0, 0)),
            pl.BlockSpec((32, 32), lambda i: (0, 0)),
            pl.BlockSpec((32, 1), lambda i: (0, 0)),
            pl.BlockSpec((32, 3), lambda i: (0, 0)),
            pl.BlockSpec((3, 1), lambda i: (0, 0)),
        ],
        out_specs=pl.BlockSpec((3, blk), lambda i: (0, i)),
    )(rows, baryT, W1, b1.reshape(32, 1), W2, b2.reshape(32, 1),
      W3, b3.reshape(3, 1))


def kernel(barycentrics, face_idx, faces, embeddings, W1, b1, W2, b2, W3, b3):
    fidx2d = face_idx.astype(jnp.int32).reshape(NW * NFCH, CH)
    f_cols = _extract_faces(faces.T)
    e_cols = _extract_emb(embeddings.T)
    rows = _sc_gather(fidx2d, *f_cols, *e_cols)
    out_t = _tc_blend_mlp(rows, barycentrics.T, W1, b1, W2, b2, W3, b3)
    return out_t.T
